# baseline (speedup 1.0000x reference)
import functools

import jax
import jax.numpy as jnp
from jax import lax
from jax.experimental import pallas as pl
from jax.experimental.pallas import tpu as pltpu
from jax.experimental.pallas import tpu_sc as plsc

N_HIDDEN = 128
N_LABEL = 1000
SEQ = 200

_NS = 16
_L = 16
_IDX_PER_S = 16
_LBL_PER_S = 64
_NG = _LBL_PER_S // _L


def _sc_fused(idx, table, W, b):
    mesh = plsc.VectorSubcoreMesh(
        core_axis_name="c", subcore_axis_name="s", num_cores=1
    )

    @functools.partial(
        pl.kernel,
        mesh=mesh,
        out_type=jax.ShapeDtypeStruct((N_LABEL,), jnp.float32),
        compiler_params=pltpu.CompilerParams(needs_layout_passes=False),
        scratch_types=[
            pltpu.VMEM((_IDX_PER_S,), jnp.int32),
            pltpu.VMEM((_IDX_PER_S, N_HIDDEN), jnp.float32),
            pltpu.VMEM((N_HIDDEN,), jnp.float32),
            pltpu.VMEM((_NS, N_HIDDEN), jnp.float32),
            pltpu.VMEM((N_HIDDEN,), jnp.float32),
            pltpu.VMEM((_LBL_PER_S, N_HIDDEN), jnp.float32),
            pltpu.VMEM((_LBL_PER_S,), jnp.float32),
            pltpu.VMEM((_LBL_PER_S,), jnp.float32),
            pltpu.VMEM_SHARED((_NS, N_HIDDEN), jnp.float32),
            pltpu.SemaphoreType.DMA,
            pltpu.SemaphoreType.DMA,
            pltpu.SemaphoreType.DMA,
        ],
    )
    def k(idx_hbm, table_hbm, w_hbm, b_hbm, out_hbm,
          idx_v, rows_v, max_v, all_v, pool_v, w_v, b_v, out_v, shared,
          sem_g, sem_w, sem_b):
        s = lax.axis_index("s")

        lbase = jnp.minimum(s * _LBL_PER_S, N_LABEL - _LBL_PER_S)
        cp_w = pltpu.async_copy(w_hbm.at[pl.ds(lbase, _LBL_PER_S)], w_v, sem_w)
        cp_b = pltpu.async_copy(b_hbm.at[pl.ds(lbase, _LBL_PER_S)], b_v, sem_b)

        ibase = jnp.minimum(s * _IDX_PER_S, SEQ - _IDX_PER_S)
        pltpu.sync_copy(idx_hbm.at[pl.ds(ibase, _IDX_PER_S)], idx_v)
        half = _IDX_PER_S // 2
        cp_g0 = pltpu.async_copy(
            table_hbm.at[idx_v.at[pl.ds(0, half)]],
            rows_v.at[pl.ds(0, half)], sem_g)
        cp_g1 = pltpu.async_copy(
            table_hbm.at[idx_v.at[pl.ds(half, half)]],
            rows_v.at[pl.ds(half, half)], sem_g)

        cp_g0.wait()
        part0 = []
        for h in range(N_HIDDEN // _L):
            sl = pl.ds(h * _L, _L)
            m = rows_v[0, sl]
            for r in range(1, half):
                m = jnp.maximum(m, rows_v[r, sl])
            part0.append(m)
        cp_g1.wait()
        for h in range(N_HIDDEN // _L):
            sl = pl.ds(h * _L, _L)
            m = part0[h]
            for r in range(half, _IDX_PER_S):
                m = jnp.maximum(m, rows_v[r, sl])
            max_v[sl] = m

        pltpu.sync_copy(max_v, shared.at[s])
        plsc.subcore_barrier()
        pltpu.sync_copy(shared, all_v)
        for h in range(N_HIDDEN // _L):
            sl = pl.ds(h * _L, _L)
            m = all_v[0, sl]
            for r in range(1, _NS):
                m = jnp.maximum(m, all_v[r, sl])
            pool_v[sl] = m

        cp_w.wait()
        cp_b.wait()
        pvecs = [pool_v[pl.ds(h * _L, _L)] for h in range(N_HIDDEN // _L)]
        dnums = lax.GatherDimensionNumbers(
            offset_dims=(), collapsed_slice_dims=(0,), start_index_map=(0,))

        def _xshuf(v, d):
            idx = (lax.iota(jnp.int32, _L) ^ d).reshape(_L, 1)
            return lax.gather(
                v, idx, dnums, slice_sizes=(1,),
                mode=lax.GatherScatterMode.PROMISE_IN_BOUNDS)

        lane = lax.iota(jnp.int32, _L)
        masks = [lane == j for j in range(_L)]
        for g in range(_NG):
            res = jnp.zeros((_L,), jnp.float32)
            for j in range(_L):
                lbl = g * _L + j
                acc = w_v[lbl, pl.ds(0, _L)] * pvecs[0]
                for h in range(1, N_HIDDEN // _L):
                    acc = acc + w_v[lbl, pl.ds(h * _L, _L)] * pvecs[h]
                for d in (1, 2, 4, 8):
                    acc = acc + _xshuf(acc, d)
                res = jnp.where(masks[j], acc, res)
            out_v[pl.ds(g * _L, _L)] = res + b_v[pl.ds(g * _L, _L)]
        pltpu.sync_copy(out_v, out_hbm.at[pl.ds(lbase, _LBL_PER_S)])

    return k(idx, table, W, b)


def kernel(x, table, W, b):
    idx = x.reshape(SEQ)
    logits = _sc_fused(idx, table, W, b)
    return logits.reshape(1, N_LABEL)


# speedup vs baseline: 1.0487x; 1.0487x over previous
import functools

import jax
import jax.numpy as jnp
from jax import lax
from jax.experimental import pallas as pl
from jax.experimental.pallas import tpu as pltpu
from jax.experimental.pallas import tpu_sc as plsc

N_HIDDEN = 128
N_LABEL = 1000
SEQ = 200

_NS = 16
_L = 16
_IDX_PER_S = 16
_LBL_PER_S = 64
_NG = _LBL_PER_S // _L


def _sc_fused(idx, table, W, b):
    mesh = plsc.VectorSubcoreMesh(
        core_axis_name="c", subcore_axis_name="s", num_cores=1
    )

    @functools.partial(
        pl.kernel,
        mesh=mesh,
        out_type=jax.ShapeDtypeStruct((N_LABEL,), jnp.float32),
        compiler_params=pltpu.CompilerParams(needs_layout_passes=False),
        scratch_types=[
            pltpu.VMEM((_IDX_PER_S,), jnp.int32),
            pltpu.VMEM((_IDX_PER_S, N_HIDDEN), jnp.float32),
            pltpu.VMEM((N_HIDDEN,), jnp.float32),
            pltpu.VMEM((_NS, N_HIDDEN), jnp.float32),
            pltpu.VMEM((N_HIDDEN,), jnp.float32),
            pltpu.VMEM((_LBL_PER_S, N_HIDDEN), jnp.float32),
            pltpu.VMEM((_LBL_PER_S,), jnp.float32),
            pltpu.VMEM((_LBL_PER_S,), jnp.float32),
            pltpu.VMEM_SHARED((_NS, N_HIDDEN), jnp.float32),
            pltpu.SemaphoreType.DMA,
            pltpu.SemaphoreType.DMA,
            pltpu.SemaphoreType.DMA,
        ],
    )
    def k(idx_hbm, table_hbm, w_hbm, b_hbm, out_hbm,
          idx_v, rows_v, max_v, all_v, pool_v, w_v, b_v, out_v, shared,
          sem_g, sem_w, sem_b):
        s = lax.axis_index("s")

        lbase = jnp.minimum(s * _LBL_PER_S, N_LABEL - _LBL_PER_S)
        cp_w = pltpu.async_copy(w_hbm.at[pl.ds(lbase, _LBL_PER_S)], w_v, sem_w)
        cp_b = pltpu.async_copy(b_hbm.at[pl.ds(lbase, _LBL_PER_S)], b_v, sem_b)

        ibase = jnp.minimum(s * _IDX_PER_S, SEQ - _IDX_PER_S)
        pltpu.sync_copy(idx_hbm.at[pl.ds(ibase, _IDX_PER_S)], idx_v)
        half = _IDX_PER_S // 2
        cp_g0 = pltpu.async_copy(
            table_hbm.at[idx_v.at[pl.ds(0, half)]],
            rows_v.at[pl.ds(0, half)], sem_g)
        cp_g1 = pltpu.async_copy(
            table_hbm.at[idx_v.at[pl.ds(half, half)]],
            rows_v.at[pl.ds(half, half)], sem_g)

        cp_g0.wait()
        part0 = []
        for h in range(N_HIDDEN // _L):
            sl = pl.ds(h * _L, _L)
            m = rows_v[0, sl]
            for r in range(1, half):
                m = jnp.maximum(m, rows_v[r, sl])
            part0.append(m)
        cp_g1.wait()
        for h in range(N_HIDDEN // _L):
            sl = pl.ds(h * _L, _L)
            m = part0[h]
            for r in range(half, _IDX_PER_S):
                m = jnp.maximum(m, rows_v[r, sl])
            max_v[sl] = m

        pltpu.sync_copy(max_v, shared.at[s])
        plsc.subcore_barrier()
        pltpu.sync_copy(shared, all_v)
        for h in range(N_HIDDEN // _L):
            sl = pl.ds(h * _L, _L)
            m = all_v[0, sl]
            for r in range(1, _NS):
                m = jnp.maximum(m, all_v[r, sl])
            pool_v[sl] = m

        cp_w.wait()
        cp_b.wait()
        pvecs = [pool_v[pl.ds(h * _L, _L)] for h in range(N_HIDDEN // _L)]
        dnums = lax.GatherDimensionNumbers(
            offset_dims=(), collapsed_slice_dims=(0,), start_index_map=(0,))

        def _xshuf(v, d):
            idx = (lax.iota(jnp.int32, _L) ^ d).reshape(_L, 1)
            return lax.gather(
                v, idx, dnums, slice_sizes=(1,),
                mode=lax.GatherScatterMode.PROMISE_IN_BOUNDS)

        lane = lax.iota(jnp.int32, _L)

        def _label_body(j, carry):
            mask = lane == j
            out = []
            for g in range(_NG):
                lbl = g * _L + j
                acc = w_v[lbl, pl.ds(0, _L)] * pvecs[0]
                for h in range(1, N_HIDDEN // _L):
                    acc = acc + w_v[lbl, pl.ds(h * _L, _L)] * pvecs[h]
                for d in (1, 2, 4, 8):
                    acc = acc + _xshuf(acc, d)
                out.append(jnp.where(mask, acc, carry[g]))
            return tuple(out)

        zero = jnp.zeros((_L,), jnp.float32)
        res = lax.fori_loop(0, _L, _label_body, (zero,) * _NG)
        for g in range(_NG):
            out_v[pl.ds(g * _L, _L)] = res[g] + b_v[pl.ds(g * _L, _L)]
        pltpu.sync_copy(out_v, out_hbm.at[pl.ds(lbase, _LBL_PER_S)])

    return k(idx, table, W, b)


def kernel(x, table, W, b):
    idx = x.reshape(SEQ)
    logits = _sc_fused(idx, table, W, b)
    return logits.reshape(1, N_LABEL)
